# baseline (speedup 1.0000x reference)
import dataclasses
import functools

import numpy as np

import jax
import jax.numpy as jnp
from jax import lax
from jax.experimental import pallas as pl
from jax.experimental.pallas import tpu as pltpu
from jax.experimental.pallas import tpu_sc as plsc

N_NODES = 10000
N_EDGES = 320000
D = 128

NC = 2
NS = 16
CH = 80
NCHUNK = N_EDGES // CH
NBUF = 4

ROWS_A = 640
ROWS_LAST = N_NODES - 15 * ROWS_A
ZCH = 40

CORE0_CHUNKS = NCHUNK // NC
N_HEAVY = 10
CPT_H = 128
CPT_L = 120


_MM_BLOCK = 1000


def _mm_first_body(x_ref, wt_ref, b_ref, o_ref):
    o_ref[...] = (
        jnp.dot(x_ref[...], wt_ref[...], preferred_element_type=jnp.float32,
                precision=lax.Precision.DEFAULT)
        + b_ref[...]
    )


def _mm_partial_body(p_ref, wt_ref, b_ref, o_ref):
    h = p_ref[0] + p_ref[1]
    o_ref[...] = (
        jnp.dot(h, wt_ref[...], preferred_element_type=jnp.float32,
                precision=lax.Precision.DEFAULT)
        + b_ref[...]
    )


def _mm_first(x, wt, b2d):
    return pl.pallas_call(
        _mm_first_body,
        grid=(N_NODES // _MM_BLOCK,),
        in_specs=[
            pl.BlockSpec((_MM_BLOCK, D), lambda i: (i, 0)),
            pl.BlockSpec((D, D), lambda i: (0, 0)),
            pl.BlockSpec((1, D), lambda i: (0, 0)),
        ],
        out_specs=pl.BlockSpec((_MM_BLOCK, D), lambda i: (i, 0)),
        out_shape=jax.ShapeDtypeStruct((N_NODES, D), jnp.float32),
    )(x, wt, b2d)


def _mm_partial(p, wt, b2d):
    return pl.pallas_call(
        _mm_partial_body,
        grid=(N_NODES // _MM_BLOCK,),
        in_specs=[
            pl.BlockSpec((NC, _MM_BLOCK, D), lambda i: (0, i, 0)),
            pl.BlockSpec((D, D), lambda i: (0, 0)),
            pl.BlockSpec((1, D), lambda i: (0, 0)),
        ],
        out_specs=pl.BlockSpec((_MM_BLOCK, D), lambda i: (i, 0)),
        out_shape=jax.ShapeDtypeStruct((N_NODES, D), jnp.float32),
    )(p, wt, b2d)


def _combine_body(p_ref, o_ref):
    o_ref[...] = p_ref[0] + p_ref[1]


def _combine(p):
    return pl.pallas_call(
        _combine_body,
        grid=(N_NODES // _MM_BLOCK,),
        in_specs=[pl.BlockSpec((NC, _MM_BLOCK, D), lambda i: (0, i, 0))],
        out_specs=pl.BlockSpec((_MM_BLOCK, D), lambda i: (i, 0)),
        out_shape=jax.ShapeDtypeStruct((N_NODES, D), jnp.float32),
    )(p)


_sc_mesh = plsc.VectorSubcoreMesh(core_axis_name="c", subcore_axis_name="s")

_sc_params = pltpu.CompilerParams()
if "needs_layout_passes" in pltpu.CompilerParams.__dataclass_fields__:
    _sc_params = dataclasses.replace(_sc_params, needs_layout_passes=False)


@functools.partial(
    pl.kernel,
    out_type=jax.ShapeDtypeStruct((NC, N_NODES, D), jnp.float32),
    mesh=_sc_mesh,
    compiler_params=_sc_params,
    scratch_types=(
        [pltpu.VMEM((CH,), jnp.int32) for _ in range(NBUF)]
        + [pltpu.VMEM((CH,), jnp.int32) for _ in range(NBUF)]
        + [pltpu.VMEM((CH,), jnp.float32) for _ in range(NBUF)]
        + [pltpu.VMEM((CH, D), jnp.float32) for _ in range(NBUF)]
        + [pltpu.VMEM_SHARED((N_NODES, D), jnp.float32)]
        + [pltpu.SemaphoreType.DMA for _ in range(4 * NBUF)]
    ),
)
def _spmm(h_hbm, src_hbm, dst_hbm, w_hbm, out_hbm, *refs):
    srcv = list(refs[0:NBUF])
    dstv = list(refs[NBUF:2 * NBUF])
    wv = list(refs[2 * NBUF:3 * NBUF])
    rows = list(refs[3 * NBUF:4 * NBUF])
    acc = refs[4 * NBUF]
    isem = list(refs[1 + 4 * NBUF:1 + 5 * NBUF])
    dsem = list(refs[1 + 5 * NBUF:1 + 6 * NBUF])
    gsem = list(refs[1 + 6 * NBUF:1 + 7 * NBUF])
    ssem = list(refs[1 + 7 * NBUF:1 + 8 * NBUF])

    c = lax.axis_index("c")
    s = lax.axis_index("s")
    heavy = s < N_HEAVY
    n_my = jnp.where(heavy, CPT_H, CPT_L)
    cbase = c * CORE0_CHUNKS + jnp.where(
        heavy, s * CPT_H, N_HEAVY * CPT_H + (s - N_HEAVY) * CPT_L)

    zvec = jnp.zeros((16,), jnp.float32)

    @pl.loop(0, ZCH)
    def _zero_rows(r):
        row = rows[0].at[r]
        for k in range(D // 16):
            row[pl.ds(k * 16, 16)] = zvec

    nz = jnp.where(s == NS - 1, ROWS_LAST // ZCH, ROWS_A // ZCH)

    @pl.loop(0, nz)
    def _zero_acc(j):
        pltpu.async_copy(
            rows[0].at[pl.ds(0, ZCH)],
            acc.at[pl.ds(s * ROWS_A + j * ZCH, ZCH)],
            gsem[0],
        )

    @pl.loop(0, nz)
    def _zero_wait(j):
        pltpu.make_async_copy(
            rows[0].at[pl.ds(0, ZCH)],
            acc.at[pl.ds(0, ZCH)],
            gsem[0],
        ).wait()

    plsc.subcore_barrier()

    def issue_idx(b, j):
        base_e = (cbase + j) * CH
        pltpu.async_copy(src_hbm.at[pl.ds(base_e, CH)], srcv[b], isem[b])
        pltpu.async_copy(w_hbm.at[pl.ds(base_e, CH)], wv[b], isem[b])

    def wait_idx(b):
        pltpu.make_async_copy(src_hbm.at[pl.ds(0, CH)], srcv[b],
                              isem[b]).wait()
        pltpu.make_async_copy(w_hbm.at[pl.ds(0, CH)], wv[b], isem[b]).wait()

    def issue_dst(b, j):
        base_e = (cbase + j) * CH
        pltpu.async_copy(dst_hbm.at[pl.ds(base_e, CH)], dstv[b], dsem[b])

    def wait_dst(b):
        pltpu.make_async_copy(src_hbm.at[pl.ds(0, CH)], dstv[b],
                              dsem[b]).wait()

    def issue_gather(b):
        pltpu.async_copy(h_hbm.at[srcv[b]], rows[b], gsem[b])

    def wait_gather(b):
        pltpu.make_async_copy(h_hbm.at[pl.ds(0, CH)], rows[b],
                              gsem[b]).wait()

    gdnums = lax.GatherDimensionNumbers(
        offset_dims=(), collapsed_slice_dims=(0,), start_index_map=(0,))

    def scale(b):
        @plsc.parallel_loop(0, CH, step=16, unroll=2)
        def _scale(g):
            wgrp = wv[b][pl.ds(g, 16)]
            for r in range(16):
                wvec = lax.gather(
                    wgrp, jnp.full((16, 1), r, jnp.int32), gdnums,
                    slice_sizes=(1,),
                    mode=lax.GatherScatterMode.PROMISE_IN_BOUNDS)
                row = rows[b].at[g + r]
                for k in range(D // 16):
                    sl = pl.ds(k * 16, 16)
                    row[sl] = row[sl] * wvec

    def issue_scatter(b, j):
        pltpu.async_copy(rows[b], acc.at[dstv[b]], ssem[b], add=True)

    def wait_scatter(b):
        pltpu.make_async_copy(rows[b], acc.at[pl.ds(0, CH)],
                              ssem[b]).wait()

    for b in range(NBUF):
        issue_idx(b, b)
        issue_dst(b, b)
    wait_idx(0)
    issue_gather(0)
    wait_idx(1)
    issue_gather(1)
    for b in range(2):
        wait_gather(b)
        wait_idx(b + 2)
        issue_gather(b + 2)
        scale(b)
        issue_idx(b, NBUF + b)
        wait_dst(b)
        issue_scatter(b, b)
    for b in range(2, 4):
        wait_gather(b)
        wait_scatter(b - 2)
        wait_idx(b - 2)
        issue_gather(b - 2)
        issue_dst(b - 2, b + 2)
        scale(b)
        issue_idx(b, NBUF + b)
        wait_dst(b)
        issue_scatter(b, b)

    @pl.loop(NBUF, n_my, step=NBUF)
    def _steady(base):
        for b in range(NBUF):
            j = base + b
            wait_gather(b)
            b2 = (b + 2) % NBUF
            wait_scatter(b2)

            @pl.when(j + 2 < n_my)
            def _prefetch():
                wait_idx(b2)
                issue_gather(b2)
                issue_dst(b2, j + 2)

            scale(b)

            @pl.when(j + NBUF < n_my)
            def _prefetch_idx():
                issue_idx(b, j + NBUF)

            wait_dst(b)
            issue_scatter(b, j)

    wait_scatter(2)
    wait_scatter(3)

    plsc.subcore_barrier()

    @pl.when(s < NS - 1)
    def _drain_a():
        pltpu.sync_copy(
            acc.at[pl.ds(s * ROWS_A, ROWS_A)],
            out_hbm.at[c].at[pl.ds(s * ROWS_A, ROWS_A)],
        )

    @pl.when(s == NS - 1)
    def _drain_last():
        pltpu.sync_copy(
            acc.at[pl.ds(15 * ROWS_A, ROWS_LAST)],
            out_hbm.at[c].at[pl.ds(15 * ROWS_A, ROWS_LAST)],
        )


def kernel(x, edge_index, edge_weight, W1, b1, W2, b2, W3, b3):
    src = edge_index[0]
    dst = edge_index[1]

    h = _mm_first(x, W1.T, b1.reshape(1, D))
    p = _spmm(h, src, dst, edge_weight)
    h = _mm_partial(p, W2.T, b2.reshape(1, D))
    p = _spmm(h, src, dst, edge_weight)
    h = _mm_partial(p, W3.T, b3.reshape(1, D))
    p = _spmm(h, src, dst, edge_weight)
    return _combine(p)


# speedup vs baseline: 1.0091x; 1.0091x over previous
import dataclasses
import functools

import numpy as np

import jax
import jax.numpy as jnp
from jax import lax
from jax.experimental import pallas as pl
from jax.experimental.pallas import tpu as pltpu
from jax.experimental.pallas import tpu_sc as plsc

N_NODES = 10000
N_EDGES = 320000
D = 128

NC = 2
NS = 16
CH = 64
NCHUNK = N_EDGES // CH
NBUF = 4

ROWS_A = 640
ROWS_LAST = N_NODES - 15 * ROWS_A
ZCH = 40

CORE0_CHUNKS = 2504
CPT_H = 160
CPT_L = 152


_MM_BLOCK = 1000


def _mm_first_body(x_ref, wt_ref, b_ref, o_ref):
    o_ref[...] = (
        jnp.dot(x_ref[...], wt_ref[...], preferred_element_type=jnp.float32,
                precision=lax.Precision.DEFAULT)
        + b_ref[...]
    )


def _mm_partial_body(p_ref, wt_ref, b_ref, o_ref):
    h = p_ref[0] + p_ref[1]
    o_ref[...] = (
        jnp.dot(h, wt_ref[...], preferred_element_type=jnp.float32,
                precision=lax.Precision.DEFAULT)
        + b_ref[...]
    )


def _mm_first(x, wt, b2d):
    return pl.pallas_call(
        _mm_first_body,
        grid=(N_NODES // _MM_BLOCK,),
        in_specs=[
            pl.BlockSpec((_MM_BLOCK, D), lambda i: (i, 0)),
            pl.BlockSpec((D, D), lambda i: (0, 0)),
            pl.BlockSpec((1, D), lambda i: (0, 0)),
        ],
        out_specs=pl.BlockSpec((_MM_BLOCK, D), lambda i: (i, 0)),
        out_shape=jax.ShapeDtypeStruct((N_NODES, D), jnp.float32),
    )(x, wt, b2d)


def _mm_partial(p, wt, b2d):
    return pl.pallas_call(
        _mm_partial_body,
        grid=(N_NODES // _MM_BLOCK,),
        in_specs=[
            pl.BlockSpec((NC, _MM_BLOCK, D), lambda i: (0, i, 0)),
            pl.BlockSpec((D, D), lambda i: (0, 0)),
            pl.BlockSpec((1, D), lambda i: (0, 0)),
        ],
        out_specs=pl.BlockSpec((_MM_BLOCK, D), lambda i: (i, 0)),
        out_shape=jax.ShapeDtypeStruct((N_NODES, D), jnp.float32),
    )(p, wt, b2d)


def _combine_body(p_ref, o_ref):
    o_ref[...] = p_ref[0] + p_ref[1]


def _combine(p):
    return pl.pallas_call(
        _combine_body,
        grid=(N_NODES // _MM_BLOCK,),
        in_specs=[pl.BlockSpec((NC, _MM_BLOCK, D), lambda i: (0, i, 0))],
        out_specs=pl.BlockSpec((_MM_BLOCK, D), lambda i: (i, 0)),
        out_shape=jax.ShapeDtypeStruct((N_NODES, D), jnp.float32),
    )(p)


_sc_mesh = plsc.VectorSubcoreMesh(core_axis_name="c", subcore_axis_name="s")

_sc_params = pltpu.CompilerParams()
if "needs_layout_passes" in pltpu.CompilerParams.__dataclass_fields__:
    _sc_params = dataclasses.replace(_sc_params, needs_layout_passes=False)


@functools.partial(
    pl.kernel,
    out_type=jax.ShapeDtypeStruct((NC, N_NODES, D), jnp.float32),
    mesh=_sc_mesh,
    compiler_params=_sc_params,
    scratch_types=(
        [pltpu.VMEM((CH,), jnp.int32) for _ in range(NBUF)]
        + [pltpu.VMEM((CH,), jnp.int32) for _ in range(NBUF)]
        + [pltpu.VMEM((CH,), jnp.float32) for _ in range(NBUF)]
        + [pltpu.VMEM((CH, D), jnp.float32) for _ in range(NBUF)]
        + [pltpu.VMEM_SHARED((N_NODES, D), jnp.float32)]
        + [pltpu.SemaphoreType.DMA for _ in range(4 * NBUF)]
    ),
)
def _spmm(h_hbm, src_hbm, dst_hbm, w_hbm, out_hbm, *refs):
    srcv = list(refs[0:NBUF])
    dstv = list(refs[NBUF:2 * NBUF])
    wv = list(refs[2 * NBUF:3 * NBUF])
    rows = list(refs[3 * NBUF:4 * NBUF])
    acc = refs[4 * NBUF]
    isem = list(refs[1 + 4 * NBUF:1 + 5 * NBUF])
    dsem = list(refs[1 + 5 * NBUF:1 + 6 * NBUF])
    gsem = list(refs[1 + 6 * NBUF:1 + 7 * NBUF])
    ssem = list(refs[1 + 7 * NBUF:1 + 8 * NBUF])

    c = lax.axis_index("c")
    s = lax.axis_index("s")
    n_heavy = jnp.where(c == 0, 9, 8)
    heavy = s < n_heavy
    n_my = jnp.where(heavy, CPT_H, CPT_L)
    cbase = c * CORE0_CHUNKS + jnp.where(
        heavy, s * CPT_H, n_heavy * CPT_H + (s - n_heavy) * CPT_L)

    zvec = jnp.zeros((16,), jnp.float32)

    @pl.loop(0, ZCH)
    def _zero_rows(r):
        row = rows[0].at[r]
        for k in range(D // 16):
            row[pl.ds(k * 16, 16)] = zvec

    nz = jnp.where(s == NS - 1, ROWS_LAST // ZCH, ROWS_A // ZCH)

    @pl.loop(0, nz)
    def _zero_acc(j):
        pltpu.async_copy(
            rows[0].at[pl.ds(0, ZCH)],
            acc.at[pl.ds(s * ROWS_A + j * ZCH, ZCH)],
            gsem[0],
        )

    @pl.loop(0, nz)
    def _zero_wait(j):
        pltpu.make_async_copy(
            rows[0].at[pl.ds(0, ZCH)],
            acc.at[pl.ds(0, ZCH)],
            gsem[0],
        ).wait()

    plsc.subcore_barrier()

    def issue_idx(b, j):
        base_e = (cbase + j) * CH
        pltpu.async_copy(src_hbm.at[pl.ds(base_e, CH)], srcv[b], isem[b])
        pltpu.async_copy(w_hbm.at[pl.ds(base_e, CH)], wv[b], isem[b])

    def wait_idx(b):
        pltpu.make_async_copy(src_hbm.at[pl.ds(0, CH)], srcv[b],
                              isem[b]).wait()
        pltpu.make_async_copy(w_hbm.at[pl.ds(0, CH)], wv[b], isem[b]).wait()

    def issue_dst(b, j):
        base_e = (cbase + j) * CH
        pltpu.async_copy(dst_hbm.at[pl.ds(base_e, CH)], dstv[b], dsem[b])

    def wait_dst(b):
        pltpu.make_async_copy(src_hbm.at[pl.ds(0, CH)], dstv[b],
                              dsem[b]).wait()

    def issue_gather(b):
        pltpu.async_copy(h_hbm.at[srcv[b]], rows[b], gsem[b])

    def wait_gather(b):
        pltpu.make_async_copy(h_hbm.at[pl.ds(0, CH)], rows[b],
                              gsem[b]).wait()

    gdnums = lax.GatherDimensionNumbers(
        offset_dims=(), collapsed_slice_dims=(0,), start_index_map=(0,))

    def scale(b):
        @plsc.parallel_loop(0, CH, step=16, unroll=2)
        def _scale(g):
            wgrp = wv[b][pl.ds(g, 16)]
            for r in range(16):
                wvec = lax.gather(
                    wgrp, jnp.full((16, 1), r, jnp.int32), gdnums,
                    slice_sizes=(1,),
                    mode=lax.GatherScatterMode.PROMISE_IN_BOUNDS)
                row = rows[b].at[g + r]
                for k in range(D // 16):
                    sl = pl.ds(k * 16, 16)
                    row[sl] = row[sl] * wvec

    def issue_scatter(b, j):
        pltpu.async_copy(rows[b], acc.at[dstv[b]], ssem[b], add=True)

    def wait_scatter(b):
        pltpu.make_async_copy(rows[b], acc.at[pl.ds(0, CH)],
                              ssem[b]).wait()

    for b in range(NBUF):
        issue_idx(b, b)
        issue_dst(b, b)
    wait_idx(0)
    issue_gather(0)
    wait_idx(1)
    issue_gather(1)
    for b in range(2):
        wait_gather(b)
        wait_idx(b + 2)
        issue_gather(b + 2)
        scale(b)
        issue_idx(b, NBUF + b)
        wait_dst(b)
        issue_scatter(b, b)
    for b in range(2, 4):
        wait_gather(b)
        wait_scatter(b - 2)
        wait_idx(b - 2)
        issue_gather(b - 2)
        issue_dst(b - 2, b + 2)
        scale(b)
        issue_idx(b, NBUF + b)
        wait_dst(b)
        issue_scatter(b, b)

    @pl.loop(NBUF, n_my, step=NBUF)
    def _steady(base):
        for b in range(NBUF):
            j = base + b
            wait_gather(b)
            b2 = (b + 2) % NBUF
            wait_scatter(b2)

            @pl.when(j + 2 < n_my)
            def _prefetch():
                wait_idx(b2)
                issue_gather(b2)
                issue_dst(b2, j + 2)

            scale(b)

            @pl.when(j + NBUF < n_my)
            def _prefetch_idx():
                issue_idx(b, j + NBUF)

            wait_dst(b)
            issue_scatter(b, j)

    wait_scatter(2)
    wait_scatter(3)

    plsc.subcore_barrier()

    @pl.when(s < NS - 1)
    def _drain_a():
        pltpu.sync_copy(
            acc.at[pl.ds(s * ROWS_A, ROWS_A)],
            out_hbm.at[c].at[pl.ds(s * ROWS_A, ROWS_A)],
        )

    @pl.when(s == NS - 1)
    def _drain_last():
        pltpu.sync_copy(
            acc.at[pl.ds(15 * ROWS_A, ROWS_LAST)],
            out_hbm.at[c].at[pl.ds(15 * ROWS_A, ROWS_LAST)],
        )


def kernel(x, edge_index, edge_weight, W1, b1, W2, b2, W3, b3):
    src = edge_index[0]
    dst = edge_index[1]

    h = _mm_first(x, W1.T, b1.reshape(1, D))
    p = _spmm(h, src, dst, edge_weight)
    h = _mm_partial(p, W2.T, b2.reshape(1, D))
    p = _spmm(h, src, dst, edge_weight)
    h = _mm_partial(p, W3.T, b3.reshape(1, D))
    p = _spmm(h, src, dst, edge_weight)
    return _combine(p)
